# baseline (speedup 1.0000x reference)
import functools

import jax
import jax.numpy as jnp
from jax import lax
from jax.experimental import pallas as pl
from jax.experimental.pallas import tpu as pltpu
from jax.experimental.pallas import tpu_sc as plsc

T, NT, NC, NE, E = 4, 20000, 50000, 50000, 100000
HD = 128
NTP, NCP = 20096, 50048
NTILES = 16
CH, CHW = 50, 128
EPT = CH * CHW
EPAD = NTILES * EPT
RT = 1000


def _sc_body(with_counts, *refs):
    if with_counts:
        (tb0, tb1, tb2, tb3, ce0, ce1, ce2, ce3,
         su_uc, sd_uc, su_he, sd_he, su_cu, sd_cu, su_eb, sd_eb,
         zrows, orows,
         sum_ce, sum_tr, cnt_ce, cnt_tr,
         acc, sidx, didx, gbuf, zob) = refs
    else:
        (tb0, tb1, tb2, tb3, ce0, ce1, ce2, ce3,
         su_uc, sd_uc, su_he, sd_he, su_cu, sd_cu, su_eb, sd_eb,
         zrows, orows,
         sum_ce, sum_tr,
         acc, sidx, didx, gbuf, zob) = refs
        cnt_ce = cnt_tr = None

    c = lax.axis_index("c")
    s = lax.axis_index("s")

    pltpu.sync_copy(zrows, zob)

    def zero_slice(start, rows_per):
        off = 0
        while off < rows_per:
            n = min(256, rows_per - off)
            pltpu.sync_copy(zob.at[pl.ds(0, n)], acc.at[pl.ds(start + off, n)])
            off += n

    def do_relation(tables, su, sd, ndp, out4, u, cnt2):
        rows_per = ndp // NTILES
        start = s * rows_per
        pltpu.sync_copy(su.at[s], sidx)
        pltpu.sync_copy(sd.at[s], didx)
        for b in range(4):
            zero_slice(start, rows_per)
            plsc.subcore_barrier()

            tbl = tables[b]

            def batch(j, _):
                pltpu.sync_copy(tbl.at[sidx.at[j]], gbuf)
                pltpu.sync_copy(gbuf, acc.at[didx.at[j]], add=True)
                return _

            lax.fori_loop(0, CH, batch, None)
            plsc.subcore_barrier()
            off = 0
            while off < rows_per:
                n = min(512, rows_per - off)
                pltpu.sync_copy(acc.at[pl.ds(start + off, n)],
                                out4.at[u, b, pl.ds(start + off, n)])
                off += n
            plsc.subcore_barrier()
        if cnt2 is not None:
            zero_slice(start, rows_per)
            pltpu.sync_copy(orows, gbuf)
            plsc.subcore_barrier()

            def cbatch(j, _):
                pltpu.sync_copy(gbuf, acc.at[didx.at[j]], add=True)
                return _

            lax.fori_loop(0, CH, cbatch, None)
            plsc.subcore_barrier()
            off = 0
            while off < rows_per:
                n = min(512, rows_per - off)
                pltpu.sync_copy(acc.at[pl.ds(start + off, n)],
                                cnt2.at[u, pl.ds(start + off, n)])
                off += n
            plsc.subcore_barrier()

    @pl.when(c == 0)
    def _():
        do_relation((tb0, tb1, tb2, tb3), su_uc, sd_uc, NCP, sum_ce, 0,
                    cnt_ce)
        do_relation((ce0, ce1, ce2, ce3), su_cu, sd_cu, NTP, sum_tr, 0,
                    cnt_tr)

    @pl.when(c == 1)
    def _():
        do_relation((tb0, tb1, tb2, tb3), su_he, sd_he, NCP, sum_ce, 1,
                    cnt_ce)
        do_relation((ce0, ce1, ce2, ce3), su_eb, sd_eb, NTP, sum_tr, 1,
                    cnt_tr)


def _make_sc_agg(with_counts):
    out_type = [
        jax.ShapeDtypeStruct((2, 4, NCP, 32), jnp.float32),
        jax.ShapeDtypeStruct((2, 4, NTP, 32), jnp.float32),
    ]
    if with_counts:
        out_type += [
            jax.ShapeDtypeStruct((2, NCP, 32), jnp.float32),
            jax.ShapeDtypeStruct((2, NTP, 32), jnp.float32),
        ]
    return pl.kernel(
        functools.partial(_sc_body, with_counts),
        out_type=out_type,
        mesh=plsc.VectorSubcoreMesh(core_axis_name="c", subcore_axis_name="s"),
        compiler_params=pltpu.CompilerParams(use_tc_tiling_on_sc=False),
        scratch_types=[
            pltpu.VMEM_SHARED((NCP, 32), jnp.float32),
            pltpu.VMEM((CH, CHW), jnp.int32),
            pltpu.VMEM((CH, CHW), jnp.int32),
            pltpu.VMEM((CHW, 32), jnp.float32),
            pltpu.VMEM((256, 32), jnp.float32),
        ],
    )


def _mm(a, b):
    return jnp.dot(a, b, preferred_element_type=jnp.float32)


def _gru0(x, wih_t, bih, bhh):
    gi = _mm(x, wih_t) + bih
    r = jax.nn.sigmoid(gi[:, :HD] + bhh[:, :HD])
    z = jax.nn.sigmoid(gi[:, HD:2 * HD] + bhh[:, HD:2 * HD])
    n = jnp.tanh(gi[:, 2 * HD:] + r * bhh[:, 2 * HD:])
    return (1.0 - z) * n


def _norm(sum_blk, cnt_blk):
    y = jnp.concatenate([sum_blk[0], sum_blk[1], sum_blk[2], sum_blk[3]],
                        axis=1)
    return y * (1.0 / jnp.maximum(cnt_blk[:, :1], 1.0))


def _pre_body(ce_ref, r1_ref, b1_ref, out_ref):
    out_ref[...] = (_mm(ce_ref[0], r1_ref[...]) + b1_ref[...])[None]


def _t1_body(x_ref, scu_ref, ccu_ref, seb_ref, ceb_ref,
             gr1_ref, gb1_ref, gw12_ref, gw13_ref,
             lr_ref, lb_ref, lw2_ref, lw3_ref,
             lwih_ref, lbih_ref, lbhh_ref, lfcw_ref, lfcb_ref, fwl_ref,
             h0_ref, h1_ref, h2_ref, h3_ref, hld_ref):
    x = x_ref[...]
    y2 = _norm(scu_ref[0], ccu_ref[0])
    y3 = _norm(seb_ref[0], ceb_ref[0])
    h1 = jax.nn.relu(_mm(x, gr1_ref[...]) + gb1_ref[...]
                     + _mm(y2, gw12_ref[...]) + _mm(y3, gw13_ref[...]))
    h0_ref[...] = h1[:, 0:32]
    h1_ref[...] = h1[:, 32:64]
    h2_ref[...] = h1[:, 64:96]
    h3_ref[...] = h1[:, 96:128]
    nf = jax.nn.relu(_mm(x, lr_ref[...]) + lb_ref[...]
                     + _mm(y2, lw2_ref[...]) + _mm(y3, lw3_ref[...]))
    hl = _gru0(nf, lwih_ref[...], lbih_ref[...], lbhh_ref[...])
    lfeat = _mm(hl, lfcw_ref[...]) + lfcb_ref[...]
    hld_ref[...] = _mm(lfeat, fwl_ref[...])


def _ce1_body(pre_ref, s_ref, c_ref, w1_ref, h0_ref, h1_ref, h2_ref, h3_ref):
    y = _norm(s_ref[0], c_ref[0])
    h1 = jax.nn.relu(pre_ref[0] + _mm(y, w1_ref[0]))
    h0_ref[...] = h1[None, :, 0:32]
    h1_ref[...] = h1[None, :, 32:64]
    h2_ref[...] = h1[None, :, 64:96]
    h3_ref[...] = h1[None, :, 96:128]


def _t2_body(h0_ref, h1_ref, h2_ref, h3_ref, scu_ref, ccu_ref, seb_ref,
             ceb_ref, gr2_ref, gb2_ref, gw22_ref, gw23_ref, mx_ref):
    h = jnp.concatenate([h0_ref[...], h1_ref[...], h2_ref[...], h3_ref[...]],
                        axis=1)
    z2 = _norm(scu_ref[0], ccu_ref[0])
    z3 = _norm(seb_ref[0], ceb_ref[0])
    h2 = jax.nn.relu(_mm(h, gr2_ref[...]) + gb2_ref[...]
                     + _mm(z2, gw22_ref[...]) + _mm(z3, gw23_ref[...]))
    tile_max = jnp.broadcast_to(jnp.max(h2, axis=0, keepdims=True), (8, HD))

    @pl.when(pl.program_id(0) == 0)
    def _():
        mx_ref[...] = jnp.zeros_like(mx_ref)

    mx_ref[...] = jnp.maximum(mx_ref[...], tile_max)


def _ce2_body(h0_ref, h1_ref, h2_ref, h3_ref, s_ref, c_ref, gr2_ref, gb2_ref,
              w2_ref, mx_ref):
    h = jnp.concatenate([h0_ref[0], h1_ref[0], h2_ref[0], h3_ref[0]], axis=1)
    z = _norm(s_ref[0], c_ref[0])
    h2 = jax.nn.relu(_mm(h, gr2_ref[...]) + gb2_ref[...] + _mm(z, w2_ref[0]))
    tile_max = jnp.broadcast_to(jnp.max(h2, axis=0, keepdims=True), (8, HD))

    @pl.when((pl.program_id(0) == 0) & (pl.program_id(1) == 0))
    def _():
        mx_ref[...] = jnp.zeros_like(mx_ref)

    mx_ref[...] = jnp.maximum(mx_ref[...], tile_max)


def _fin_body(mt_ref, mce_ref, hld_ref,
              gwih_ref, gbih_ref, gbhh_ref, gfcw_ref, gfcb_ref, fwg_ref,
              fb_ref, out_ref):
    m = jnp.maximum(jnp.max(mt_ref[0], axis=0, keepdims=True),
                    jnp.max(mce_ref[0], axis=0, keepdims=True))
    hg = _gru0(m, gwih_ref[...], gbih_ref[...], gbhh_ref[...])
    gfeat = _mm(hg, gfcw_ref[...]) + gfcb_ref[...]
    sg = _mm(gfeat, fwg_ref[...]) + fb_ref[...]
    out_ref[...] = jax.nn.sigmoid(hld_ref[...] + sg[0, 0])


def kernel(trans_x, uc_src, uc_dst, he_src, he_dst, cu_src, cu_dst,
           eb_src, eb_dst, card_emb, email_emb,
           g_W1, g_R1, g_b1, g_W2, g_R2, g_b2, g_Wih, g_Whh, g_bih, g_bhh,
           g_fcW, g_fcb, l_W, l_R, l_b, l_Wih, l_Whh, l_bih, l_bhh,
           l_fcW, l_fcb, f_W, f_b):
    f32 = jnp.float32
    i32 = jnp.int32

    xt32 = trans_x.reshape(T, NT, 4, 32).transpose(0, 2, 1, 3)
    ce = jnp.stack([card_emb, email_emb])
    ce32 = ce.reshape(2, NC, 4, 32).transpose(2, 0, 1, 3).reshape(4, 2 * NC, 32)

    def prep_idx(src, dst, dump, src_off=0):
        sp = jnp.pad(src.astype(i32) + src_off, ((0, 0), (0, EPAD - E)),
                     constant_values=src_off)
        dp = jnp.pad(dst.astype(i32), ((0, 0), (0, EPAD - E)),
                     constant_values=dump)
        return (sp.reshape(T, NTILES, CH, CHW),
                dp.reshape(T, NTILES, CH, CHW))

    su_uc, sd_uc = prep_idx(uc_src, uc_dst, NC)
    su_he, sd_he = prep_idx(he_src, he_dst, NE)
    su_cu, sd_cu = prep_idx(cu_src, cu_dst, NT)
    su_eb, sd_eb = prep_idx(eb_src, eb_dst, NT, src_off=NC)

    zrows = jnp.zeros((256, 32), f32)
    orows = jnp.ones((CHW, 32), f32)

    gb1 = g_b1[None]; gb2 = g_b2[None]; lb = l_b[None]
    gw12, gw13 = g_W1[2], g_W1[3]
    gw22, gw23 = g_W2[2], g_W2[3]
    lw2, lw3 = l_W[2], l_W[3]
    lwih_t = l_Wih.T; gwih_t = g_Wih.T
    lbih = l_bih[None]; lbhh = l_bhh[None]
    gbih = g_bih[None]; gbhh = g_bhh[None]
    lfcw_t = l_fcW.T; gfcw_t = g_fcW.T
    lfcb = l_fcb[None]; gfcb = g_fcb[None]
    fwl = f_W[0, HD:][:, None]; fwg = f_W[0, :HD][:, None]
    fb = f_b[None]

    sc1 = _make_sc_agg(True)
    sc2 = _make_sc_agg(False)

    wspec = pl.BlockSpec((HD, HD), lambda i: (0, 0))
    bspec = pl.BlockSpec((1, HD), lambda i: (0, 0))
    b3spec = pl.BlockSpec((1, 3 * HD), lambda i: (0, 0))
    w3spec = pl.BlockSpec((HD, 3 * HD), lambda i: (0, 0))
    vspec = pl.BlockSpec((HD, 1), lambda i: (0, 0))
    xspec = pl.BlockSpec((RT, HD), lambda i: (i, 0))
    sum_ce_spec = pl.BlockSpec((1, 4, RT, 32), lambda u, i: (u, 0, i, 0))
    cnt_ce_spec = pl.BlockSpec((1, RT, 32), lambda u, i: (u, i, 0))
    sum_tr_spec = lambda u: pl.BlockSpec((1, 4, RT, 32),
                                         lambda i: (u, 0, i, 0))
    cnt_tr_spec = lambda u: pl.BlockSpec((1, RT, 32), lambda i: (u, i, 0))
    h32_spec = pl.BlockSpec((RT, 32), lambda i: (i, 0))
    hce_spec = pl.BlockSpec((1, RT, 32), lambda u, i: (u, i, 0))
    pre_spec = pl.BlockSpec((1, RT, HD), lambda u, i: (u, i, 0))
    wsel_spec = pl.BlockSpec((1, HD, HD), lambda u, i: (u, 0, 0))
    wfull2 = pl.BlockSpec((HD, HD), lambda u, i: (0, 0))
    bfull2 = pl.BlockSpec((1, HD), lambda u, i: (0, 0))
    mx_spec = pl.BlockSpec((8, HD), lambda i: (0, 0))
    mx2_spec = pl.BlockSpec((8, HD), lambda u, i: (0, 0))

    pre_ce = pl.pallas_call(
        _pre_body,
        grid=(2, NC // RT),
        in_specs=[pre_spec, wfull2, bfull2],
        out_specs=pre_spec,
        out_shape=jax.ShapeDtypeStruct((2, NC, HD), f32),
    )(ce, g_R1, gb1)

    t1_call = pl.pallas_call(
        _t1_body,
        grid=(NT // RT,),
        in_specs=[xspec, sum_tr_spec(0), cnt_tr_spec(0), sum_tr_spec(1),
                  cnt_tr_spec(1), wspec, bspec, wspec, wspec,
                  wspec, bspec, wspec, wspec,
                  w3spec, b3spec, b3spec, wspec, bspec, vspec],
        out_specs=[h32_spec, h32_spec, h32_spec, h32_spec,
                   pl.BlockSpec((RT, 1), lambda i: (i, 0))],
        out_shape=[jax.ShapeDtypeStruct((NT, 32), f32)] * 4
        + [jax.ShapeDtypeStruct((NT, 1), f32)],
    )

    ce1_call = pl.pallas_call(
        _ce1_body,
        grid=(2, NC // RT),
        in_specs=[pre_spec, sum_ce_spec, cnt_ce_spec, wsel_spec],
        out_specs=[hce_spec] * 4,
        out_shape=[jax.ShapeDtypeStruct((2, NC, 32), f32)] * 4,
    )

    t2_call = pl.pallas_call(
        _t2_body,
        grid=(NT // RT,),
        in_specs=[h32_spec, h32_spec, h32_spec, h32_spec,
                  sum_tr_spec(0), cnt_tr_spec(0), sum_tr_spec(1),
                  cnt_tr_spec(1), wspec, bspec, wspec, wspec],
        out_specs=mx_spec,
        out_shape=jax.ShapeDtypeStruct((8, HD), f32),
        compiler_params=pltpu.CompilerParams(
            dimension_semantics=("arbitrary",)),
    )

    ce2_call = pl.pallas_call(
        _ce2_body,
        grid=(2, NC // RT),
        in_specs=[hce_spec, hce_spec, hce_spec, hce_spec,
                  sum_ce_spec, cnt_ce_spec, wfull2, bfull2, wsel_spec],
        out_specs=mx2_spec,
        out_shape=jax.ShapeDtypeStruct((8, HD), f32),
        compiler_params=pltpu.CompilerParams(
            dimension_semantics=("arbitrary", "arbitrary")),
    )

    mts, mces, hlds = [], [], []
    for t in range(T):
        tb = [xt32[t, b] for b in range(4)]
        cearg = [ce32[b] for b in range(4)]
        idx = (su_uc[t], sd_uc[t], su_he[t], sd_he[t],
               su_cu[t], sd_cu[t], su_eb[t], sd_eb[t])
        sum_ce1, sum_tr1, cnt_ce, cnt_tr = sc1(
            *tb, *cearg, *idx, zrows, orows)
        h1t = t1_call(trans_x[t], sum_tr1, cnt_tr, sum_tr1, cnt_tr,
                      g_R1, gb1, gw12, gw13, l_R, lb, lw2, lw3,
                      lwih_t, lbih, lbhh, lfcw_t, lfcb, fwl)
        h1t_b, hld = h1t[:4], h1t[4]
        h1ce_b = ce1_call(pre_ce, sum_ce1, cnt_ce, g_W1)
        h1ce_flat = [a.reshape(2 * NC, 32) for a in h1ce_b]
        sum_ce2, sum_tr2 = sc2(*h1t_b, *h1ce_flat, *idx, zrows, orows)
        mt = t2_call(*h1t_b, sum_tr2, cnt_tr, sum_tr2, cnt_tr,
                     g_R2, gb2, gw22, gw23)
        mce = ce2_call(*h1ce_b, sum_ce2, cnt_ce, g_R2, gb2, g_W2)
        mts.append(mt); mces.append(mce); hlds.append(hld)

    mt_all = jnp.stack(mts)
    mce_all = jnp.stack(mces)
    hld_all = jnp.stack(hlds)

    preds = pl.pallas_call(
        _fin_body,
        grid=(T, NT // RT),
        in_specs=[pl.BlockSpec((1, 8, HD), lambda t, i: (t, 0, 0)),
                  pl.BlockSpec((1, 8, HD), lambda t, i: (t, 0, 0)),
                  pl.BlockSpec((1, RT, 1), lambda t, i: (t, i, 0)),
                  pl.BlockSpec((HD, 3 * HD), lambda t, i: (0, 0)),
                  pl.BlockSpec((1, 3 * HD), lambda t, i: (0, 0)),
                  pl.BlockSpec((1, 3 * HD), lambda t, i: (0, 0)),
                  pl.BlockSpec((HD, HD), lambda t, i: (0, 0)),
                  pl.BlockSpec((1, HD), lambda t, i: (0, 0)),
                  pl.BlockSpec((HD, 1), lambda t, i: (0, 0)),
                  pl.BlockSpec((1, 1), lambda t, i: (0, 0))],
        out_specs=pl.BlockSpec((1, RT, 1), lambda t, i: (t, i, 0)),
        out_shape=jax.ShapeDtypeStruct((T, NT, 1), f32),
    )(mt_all, mce_all, hld_all, gwih_t, gbih, gbhh, gfcw_t, gfcb, fwg, fb)

    return preds.reshape(-1)


# speedup vs baseline: 1.3480x; 1.3480x over previous
import functools

import jax
import jax.numpy as jnp
from jax import lax
from jax.experimental import pallas as pl
from jax.experimental.pallas import tpu as pltpu
from jax.experimental.pallas import tpu_sc as plsc

T, NT, NC, NE, E = 4, 20000, 50000, 50000, 100000
HD = 128
NTP, NCP = 20096, 50048
NTILES = 16
CH, CHW = 50, 128
EPT = CH * CHW
EPAD = NTILES * EPT
RT = 1000


def _sc_body(with_counts, *refs):
    if with_counts:
        (tb0, tb1, tb2, tb3, ce0, ce1, ce2, ce3,
         su_uc, sd_uc, su_he, sd_he, su_cu, sd_cu, su_eb, sd_eb,
         zrows, orows,
         sum_ce, sum_tr, cnt_ce, cnt_tr,
         acc, sidx, didx, gbuf, zob) = refs
    else:
        (tb0, tb1, tb2, tb3, ce0, ce1, ce2, ce3,
         su_uc, sd_uc, su_he, sd_he, su_cu, sd_cu, su_eb, sd_eb,
         zrows, orows,
         sum_ce, sum_tr,
         acc, sidx, didx, gbuf, zob) = refs
        cnt_ce = cnt_tr = None

    c = lax.axis_index("c")
    s = lax.axis_index("s")

    pltpu.sync_copy(zrows, zob)

    def zero_slice(start, rows_per):
        off = 0
        while off < rows_per:
            n = min(256, rows_per - off)
            pltpu.sync_copy(zob.at[pl.ds(0, n)], acc.at[pl.ds(start + off, n)])
            off += n

    def do_relation(tables, su, sd, ndp, out4, u, cnt2):
        rows_per = ndp // NTILES
        start = s * rows_per
        pltpu.sync_copy(su.at[s], sidx)
        pltpu.sync_copy(sd.at[s], didx)
        for b in range(4):
            zero_slice(start, rows_per)
            plsc.subcore_barrier()

            tbl = tables[b]

            def batch(j, _):
                pltpu.sync_copy(tbl.at[sidx.at[j]], gbuf)
                pltpu.sync_copy(gbuf, acc.at[didx.at[j]], add=True)
                return _

            lax.fori_loop(0, CH, batch, None)
            plsc.subcore_barrier()
            off = 0
            while off < rows_per:
                n = min(512, rows_per - off)
                pltpu.sync_copy(acc.at[pl.ds(start + off, n)],
                                out4.at[u, b, pl.ds(start + off, n)])
                off += n
            plsc.subcore_barrier()
        if cnt2 is not None:
            zero_slice(start, rows_per)
            pltpu.sync_copy(orows, gbuf)
            plsc.subcore_barrier()

            def cbatch(j, _):
                pltpu.sync_copy(gbuf, acc.at[didx.at[j]], add=True)
                return _

            lax.fori_loop(0, CH, cbatch, None)
            plsc.subcore_barrier()
            off = 0
            while off < rows_per:
                n = min(512, rows_per - off)
                pltpu.sync_copy(acc.at[pl.ds(start + off, n)],
                                cnt2.at[u, pl.ds(start + off, n)])
                off += n
            plsc.subcore_barrier()

    @pl.when(c == 0)
    def _():
        do_relation((tb0, tb1, tb2, tb3), su_uc, sd_uc, NCP, sum_ce, 0,
                    cnt_ce)
        do_relation((ce0, ce1, ce2, ce3), su_cu, sd_cu, NTP, sum_tr, 0,
                    cnt_tr)

    @pl.when(c == 1)
    def _():
        do_relation((tb0, tb1, tb2, tb3), su_he, sd_he, NCP, sum_ce, 1,
                    cnt_ce)
        do_relation((ce0, ce1, ce2, ce3), su_eb, sd_eb, NTP, sum_tr, 1,
                    cnt_tr)


def _make_sc_agg(with_counts):
    out_type = [
        jax.ShapeDtypeStruct((2, 4, NCP, 32), jnp.float32),
        jax.ShapeDtypeStruct((2, 4, NTP, 32), jnp.float32),
    ]
    if with_counts:
        out_type += [
            jax.ShapeDtypeStruct((2, NCP, 32), jnp.float32),
            jax.ShapeDtypeStruct((2, NTP, 32), jnp.float32),
        ]
    return pl.kernel(
        functools.partial(_sc_body, with_counts),
        out_type=out_type,
        mesh=plsc.VectorSubcoreMesh(core_axis_name="c", subcore_axis_name="s"),
        compiler_params=pltpu.CompilerParams(use_tc_tiling_on_sc=False),
        scratch_types=[
            pltpu.VMEM_SHARED((NCP, 32), jnp.float32),
            pltpu.VMEM((CH, CHW), jnp.int32),
            pltpu.VMEM((CH, CHW), jnp.int32),
            pltpu.VMEM((CHW, 32), jnp.float32),
            pltpu.VMEM((256, 32), jnp.float32),
        ],
    )


def _mm(a, b):
    return jnp.dot(a, b, preferred_element_type=jnp.float32)


def _gru0(x, wih_t, bih, bhh):
    gi = _mm(x, wih_t) + bih
    r = jax.nn.sigmoid(gi[:, :HD] + bhh[:, :HD])
    z = jax.nn.sigmoid(gi[:, HD:2 * HD] + bhh[:, HD:2 * HD])
    n = jnp.tanh(gi[:, 2 * HD:] + r * bhh[:, 2 * HD:])
    return (1.0 - z) * n


def _norm(sum_blk, cnt_blk):
    y = jnp.concatenate([sum_blk[0], sum_blk[1], sum_blk[2], sum_blk[3]],
                        axis=1)
    return y * (1.0 / jnp.maximum(cnt_blk[:, :1], 1.0))


def _pre_body(ce_ref, r1_ref, b1_ref, out_ref):
    out_ref[...] = (_mm(ce_ref[0], r1_ref[...]) + b1_ref[...])[None]


def _t1_body(x_ref, scu_ref, ccu_ref, seb_ref, ceb_ref,
             gr1_ref, gb1_ref, gw12_ref, gw13_ref,
             lr_ref, lb_ref, lw2_ref, lw3_ref,
             lwih_ref, lbih_ref, lbhh_ref, lfcw_ref, lfcb_ref, fwl_ref,
             h0_ref, h1_ref, h2_ref, h3_ref, hld_ref):
    x = x_ref[...]
    y2 = _norm(scu_ref[0], ccu_ref[0])
    y3 = _norm(seb_ref[0], ceb_ref[0])
    h1 = jax.nn.relu(_mm(x, gr1_ref[...]) + gb1_ref[...]
                     + _mm(y2, gw12_ref[...]) + _mm(y3, gw13_ref[...]))
    h0_ref[...] = h1[:, 0:32]
    h1_ref[...] = h1[:, 32:64]
    h2_ref[...] = h1[:, 64:96]
    h3_ref[...] = h1[:, 96:128]
    nf = jax.nn.relu(_mm(x, lr_ref[...]) + lb_ref[...]
                     + _mm(y2, lw2_ref[...]) + _mm(y3, lw3_ref[...]))
    hl = _gru0(nf, lwih_ref[...], lbih_ref[...], lbhh_ref[...])
    lfeat = _mm(hl, lfcw_ref[...]) + lfcb_ref[...]
    hld_ref[...] = _mm(lfeat, fwl_ref[...])


def _ce1_body(pre_ref, s_ref, c_ref, w1_ref, h0_ref, h1_ref, h2_ref, h3_ref):
    y = _norm(s_ref[0], c_ref[0])
    h1 = jax.nn.relu(pre_ref[0] + _mm(y, w1_ref[0]))
    h0_ref[...] = h1[None, :, 0:32]
    h1_ref[...] = h1[None, :, 32:64]
    h2_ref[...] = h1[None, :, 64:96]
    h3_ref[...] = h1[None, :, 96:128]


def _t2_body(h0_ref, h1_ref, h2_ref, h3_ref, scu_ref, ccu_ref, seb_ref,
             ceb_ref, gr2_ref, gb2_ref, gw22_ref, gw23_ref, mx_ref):
    h = jnp.concatenate([h0_ref[...], h1_ref[...], h2_ref[...], h3_ref[...]],
                        axis=1)
    z2 = _norm(scu_ref[0], ccu_ref[0])
    z3 = _norm(seb_ref[0], ceb_ref[0])
    h2 = jax.nn.relu(_mm(h, gr2_ref[...]) + gb2_ref[...]
                     + _mm(z2, gw22_ref[...]) + _mm(z3, gw23_ref[...]))
    tile_max = jnp.broadcast_to(jnp.max(h2, axis=0, keepdims=True), (8, HD))

    @pl.when(pl.program_id(0) == 0)
    def _():
        mx_ref[...] = jnp.zeros_like(mx_ref)

    mx_ref[...] = jnp.maximum(mx_ref[...], tile_max)


def _ce2_body(h0_ref, h1_ref, h2_ref, h3_ref, s_ref, c_ref, gr2_ref, gb2_ref,
              w2_ref, mx_ref):
    h = jnp.concatenate([h0_ref[0], h1_ref[0], h2_ref[0], h3_ref[0]], axis=1)
    z = _norm(s_ref[0], c_ref[0])
    h2 = jax.nn.relu(_mm(h, gr2_ref[...]) + gb2_ref[...] + _mm(z, w2_ref[0]))
    tile_max = jnp.broadcast_to(jnp.max(h2, axis=0, keepdims=True), (8, HD))

    @pl.when((pl.program_id(0) == 0) & (pl.program_id(1) == 0))
    def _():
        mx_ref[...] = jnp.zeros_like(mx_ref)

    mx_ref[...] = jnp.maximum(mx_ref[...], tile_max)


def _fin_body(mt_ref, mce_ref, hld_ref,
              gwih_ref, gbih_ref, gbhh_ref, gfcw_ref, gfcb_ref, fwg_ref,
              fb_ref, out_ref):
    m = jnp.maximum(jnp.max(mt_ref[0], axis=0, keepdims=True),
                    jnp.max(mce_ref[0], axis=0, keepdims=True))
    hg = _gru0(m, gwih_ref[...], gbih_ref[...], gbhh_ref[...])
    gfeat = _mm(hg, gfcw_ref[...]) + gfcb_ref[...]
    sg = _mm(gfeat, fwg_ref[...]) + fb_ref[...]
    out_ref[...] = jax.nn.sigmoid(hld_ref[...] + sg[0, 0])


def kernel(trans_x, uc_src, uc_dst, he_src, he_dst, cu_src, cu_dst,
           eb_src, eb_dst, card_emb, email_emb,
           g_W1, g_R1, g_b1, g_W2, g_R2, g_b2, g_Wih, g_Whh, g_bih, g_bhh,
           g_fcW, g_fcb, l_W, l_R, l_b, l_Wih, l_Whh, l_bih, l_bhh,
           l_fcW, l_fcb, f_W, f_b):
    f32 = jnp.float32
    i32 = jnp.int32

    xt32 = trans_x.reshape(T, NT, 4, 32).transpose(0, 2, 1, 3)
    ce = jnp.stack([card_emb, email_emb])
    ce32 = ce.reshape(2, NC, 4, 32).transpose(2, 0, 1, 3).reshape(4, 2 * NC, 32)

    def prep_idx(src, dst, dump, src_off=0):
        pad_src = src_off + (jnp.arange(EPAD - E, dtype=i32) % 32)
        pad_dst = dump + (jnp.arange(EPAD - E, dtype=i32) % 32)
        sp = jnp.concatenate(
            [src.astype(i32) + src_off,
             jnp.broadcast_to(pad_src, (T, EPAD - E))], axis=1)
        dp = jnp.concatenate(
            [dst.astype(i32), jnp.broadcast_to(pad_dst, (T, EPAD - E))],
            axis=1)
        return (sp.reshape(T, NTILES, CH, CHW),
                dp.reshape(T, NTILES, CH, CHW))

    su_uc, sd_uc = prep_idx(uc_src, uc_dst, NC)
    su_he, sd_he = prep_idx(he_src, he_dst, NE)
    su_cu, sd_cu = prep_idx(cu_src, cu_dst, NT)
    su_eb, sd_eb = prep_idx(eb_src, eb_dst, NT, src_off=NC)

    zrows = jnp.zeros((256, 32), f32)
    orows = jnp.ones((CHW, 32), f32)

    gb1 = g_b1[None]; gb2 = g_b2[None]; lb = l_b[None]
    gw12, gw13 = g_W1[2], g_W1[3]
    gw22, gw23 = g_W2[2], g_W2[3]
    lw2, lw3 = l_W[2], l_W[3]
    lwih_t = l_Wih.T; gwih_t = g_Wih.T
    lbih = l_bih[None]; lbhh = l_bhh[None]
    gbih = g_bih[None]; gbhh = g_bhh[None]
    lfcw_t = l_fcW.T; gfcw_t = g_fcW.T
    lfcb = l_fcb[None]; gfcb = g_fcb[None]
    fwl = f_W[0, HD:][:, None]; fwg = f_W[0, :HD][:, None]
    fb = f_b[None]

    sc1 = _make_sc_agg(True)
    sc2 = _make_sc_agg(False)

    wspec = pl.BlockSpec((HD, HD), lambda i: (0, 0))
    bspec = pl.BlockSpec((1, HD), lambda i: (0, 0))
    b3spec = pl.BlockSpec((1, 3 * HD), lambda i: (0, 0))
    w3spec = pl.BlockSpec((HD, 3 * HD), lambda i: (0, 0))
    vspec = pl.BlockSpec((HD, 1), lambda i: (0, 0))
    xspec = pl.BlockSpec((RT, HD), lambda i: (i, 0))
    sum_ce_spec = pl.BlockSpec((1, 4, RT, 32), lambda u, i: (u, 0, i, 0))
    cnt_ce_spec = pl.BlockSpec((1, RT, 32), lambda u, i: (u, i, 0))
    sum_tr_spec = lambda u: pl.BlockSpec((1, 4, RT, 32),
                                         lambda i: (u, 0, i, 0))
    cnt_tr_spec = lambda u: pl.BlockSpec((1, RT, 32), lambda i: (u, i, 0))
    h32_spec = pl.BlockSpec((RT, 32), lambda i: (i, 0))
    hce_spec = pl.BlockSpec((1, RT, 32), lambda u, i: (u, i, 0))
    pre_spec = pl.BlockSpec((1, RT, HD), lambda u, i: (u, i, 0))
    wsel_spec = pl.BlockSpec((1, HD, HD), lambda u, i: (u, 0, 0))
    wfull2 = pl.BlockSpec((HD, HD), lambda u, i: (0, 0))
    bfull2 = pl.BlockSpec((1, HD), lambda u, i: (0, 0))
    mx_spec = pl.BlockSpec((8, HD), lambda i: (0, 0))
    mx2_spec = pl.BlockSpec((8, HD), lambda u, i: (0, 0))

    pre_ce = pl.pallas_call(
        _pre_body,
        grid=(2, NC // RT),
        in_specs=[pre_spec, wfull2, bfull2],
        out_specs=pre_spec,
        out_shape=jax.ShapeDtypeStruct((2, NC, HD), f32),
    )(ce, g_R1, gb1)

    t1_call = pl.pallas_call(
        _t1_body,
        grid=(NT // RT,),
        in_specs=[xspec, sum_tr_spec(0), cnt_tr_spec(0), sum_tr_spec(1),
                  cnt_tr_spec(1), wspec, bspec, wspec, wspec,
                  wspec, bspec, wspec, wspec,
                  w3spec, b3spec, b3spec, wspec, bspec, vspec],
        out_specs=[h32_spec, h32_spec, h32_spec, h32_spec,
                   pl.BlockSpec((RT, 1), lambda i: (i, 0))],
        out_shape=[jax.ShapeDtypeStruct((NT, 32), f32)] * 4
        + [jax.ShapeDtypeStruct((NT, 1), f32)],
    )

    ce1_call = pl.pallas_call(
        _ce1_body,
        grid=(2, NC // RT),
        in_specs=[pre_spec, sum_ce_spec, cnt_ce_spec, wsel_spec],
        out_specs=[hce_spec] * 4,
        out_shape=[jax.ShapeDtypeStruct((2, NC, 32), f32)] * 4,
    )

    t2_call = pl.pallas_call(
        _t2_body,
        grid=(NT // RT,),
        in_specs=[h32_spec, h32_spec, h32_spec, h32_spec,
                  sum_tr_spec(0), cnt_tr_spec(0), sum_tr_spec(1),
                  cnt_tr_spec(1), wspec, bspec, wspec, wspec],
        out_specs=mx_spec,
        out_shape=jax.ShapeDtypeStruct((8, HD), f32),
        compiler_params=pltpu.CompilerParams(
            dimension_semantics=("arbitrary",)),
    )

    ce2_call = pl.pallas_call(
        _ce2_body,
        grid=(2, NC // RT),
        in_specs=[hce_spec, hce_spec, hce_spec, hce_spec,
                  sum_ce_spec, cnt_ce_spec, wfull2, bfull2, wsel_spec],
        out_specs=mx2_spec,
        out_shape=jax.ShapeDtypeStruct((8, HD), f32),
        compiler_params=pltpu.CompilerParams(
            dimension_semantics=("arbitrary", "arbitrary")),
    )

    mts, mces, hlds = [], [], []
    for t in range(T):
        tb = [xt32[t, b] for b in range(4)]
        cearg = [ce32[b] for b in range(4)]
        idx = (su_uc[t], sd_uc[t], su_he[t], sd_he[t],
               su_cu[t], sd_cu[t], su_eb[t], sd_eb[t])
        sum_ce1, sum_tr1, cnt_ce, cnt_tr = sc1(
            *tb, *cearg, *idx, zrows, orows)
        h1t = t1_call(trans_x[t], sum_tr1, cnt_tr, sum_tr1, cnt_tr,
                      g_R1, gb1, gw12, gw13, l_R, lb, lw2, lw3,
                      lwih_t, lbih, lbhh, lfcw_t, lfcb, fwl)
        h1t_b, hld = h1t[:4], h1t[4]
        h1ce_b = ce1_call(pre_ce, sum_ce1, cnt_ce, g_W1)
        h1ce_flat = [a.reshape(2 * NC, 32) for a in h1ce_b]
        sum_ce2, sum_tr2 = sc2(*h1t_b, *h1ce_flat, *idx, zrows, orows)
        mt = t2_call(*h1t_b, sum_tr2, cnt_tr, sum_tr2, cnt_tr,
                     g_R2, gb2, gw22, gw23)
        mce = ce2_call(*h1ce_b, sum_ce2, cnt_ce, g_R2, gb2, g_W2)
        mts.append(mt); mces.append(mce); hlds.append(hld)

    mt_all = jnp.stack(mts)
    mce_all = jnp.stack(mces)
    hld_all = jnp.stack(hlds)

    preds = pl.pallas_call(
        _fin_body,
        grid=(T, NT // RT),
        in_specs=[pl.BlockSpec((1, 8, HD), lambda t, i: (t, 0, 0)),
                  pl.BlockSpec((1, 8, HD), lambda t, i: (t, 0, 0)),
                  pl.BlockSpec((1, RT, 1), lambda t, i: (t, i, 0)),
                  pl.BlockSpec((HD, 3 * HD), lambda t, i: (0, 0)),
                  pl.BlockSpec((1, 3 * HD), lambda t, i: (0, 0)),
                  pl.BlockSpec((1, 3 * HD), lambda t, i: (0, 0)),
                  pl.BlockSpec((HD, HD), lambda t, i: (0, 0)),
                  pl.BlockSpec((1, HD), lambda t, i: (0, 0)),
                  pl.BlockSpec((HD, 1), lambda t, i: (0, 0)),
                  pl.BlockSpec((1, 1), lambda t, i: (0, 0))],
        out_specs=pl.BlockSpec((1, RT, 1), lambda t, i: (t, i, 0)),
        out_shape=jax.ShapeDtypeStruct((T, NT, 1), f32),
    )(mt_all, mce_all, hld_all, gwih_t, gbih, gbhh, gfcw_t, gfcb, fwg, fb)

    return preds.reshape(-1)


# speedup vs baseline: 1.5157x; 1.1243x over previous
import functools

import jax
import jax.numpy as jnp
from jax import lax
from jax.experimental import pallas as pl
from jax.experimental.pallas import tpu as pltpu
from jax.experimental.pallas import tpu_sc as plsc

T, NT, NC, NE, E = 4, 20000, 50000, 50000, 100000
HD = 128
NTP, NCP = 20096, 50048
NTILES = 16
CH, CHW = 50, 128
EPT = CH * CHW
EPAD = NTILES * EPT
RT = 1000


def _sc_body(with_counts, *refs):
    if with_counts:
        (tb0, tb1, tb2, tb3, ce0, ce1, ce2, ce3,
         su_uc, sd_uc, su_he, sd_he, su_cu, sd_cu, su_eb, sd_eb,
         zrows, orows,
         sum_ce, sum_tr, cnt_ce, cnt_tr,
         acc, sidx, didx, gbuf, gbuf2, zob, sg0, sg1) = refs
    else:
        (tb0, tb1, tb2, tb3, ce0, ce1, ce2, ce3,
         su_uc, sd_uc, su_he, sd_he, su_cu, sd_cu, su_eb, sd_eb,
         zrows, orows,
         sum_ce, sum_tr,
         acc, sidx, didx, gbuf, gbuf2, zob, sg0, sg1) = refs
        cnt_ce = cnt_tr = None

    c = lax.axis_index("c")
    s = lax.axis_index("s")

    pltpu.sync_copy(zrows, zob)

    def zero_slice(start, rows_per):
        off = 0
        while off < rows_per:
            n = min(256, rows_per - off)
            pltpu.sync_copy(zob.at[pl.ds(0, n)], acc.at[pl.ds(start + off, n)])
            off += n

    def do_relation(tables, su, sd, ndp, out4, u, cnt2):
        rows_per = ndp // NTILES
        start = s * rows_per
        pltpu.sync_copy(su.at[s], sidx)
        pltpu.sync_copy(sd.at[s], didx)
        for b in range(4):
            zero_slice(start, rows_per)
            plsc.subcore_barrier()

            tbl = tables[b]
            pltpu.async_copy(tbl.at[sidx.at[0]], gbuf, sg0)
            pltpu.async_copy(tbl.at[sidx.at[1]], gbuf2, sg1)

            def pair(j2, _):
                j0 = 2 * j2
                pltpu.make_async_copy(tbl.at[sidx.at[j0]], gbuf, sg0).wait()
                pltpu.sync_copy(gbuf, acc.at[didx.at[j0]], add=True)
                pltpu.async_copy(
                    tbl.at[sidx.at[jnp.minimum(j0 + 2, CH - 1)]], gbuf, sg0)
                pltpu.make_async_copy(tbl.at[sidx.at[j0]], gbuf2, sg1).wait()
                pltpu.sync_copy(gbuf2, acc.at[didx.at[j0 + 1]], add=True)
                pltpu.async_copy(
                    tbl.at[sidx.at[jnp.minimum(j0 + 3, CH - 1)]], gbuf2, sg1)
                return _

            lax.fori_loop(0, CH // 2, pair, None)
            pltpu.make_async_copy(tbl.at[sidx.at[0]], gbuf, sg0).wait()
            pltpu.make_async_copy(tbl.at[sidx.at[0]], gbuf2, sg1).wait()
            plsc.subcore_barrier()
            off = 0
            while off < rows_per:
                n = min(512, rows_per - off)
                pltpu.sync_copy(acc.at[pl.ds(start + off, n)],
                                out4.at[u, b, pl.ds(start + off, n)])
                off += n
            plsc.subcore_barrier()
        if cnt2 is not None:
            zero_slice(start, rows_per)
            pltpu.sync_copy(orows, gbuf)
            plsc.subcore_barrier()

            def cbatch(j, _):
                pltpu.sync_copy(gbuf, acc.at[didx.at[j]], add=True)
                return _

            lax.fori_loop(0, CH, cbatch, None)
            plsc.subcore_barrier()
            off = 0
            while off < rows_per:
                n = min(512, rows_per - off)
                pltpu.sync_copy(acc.at[pl.ds(start + off, n)],
                                cnt2.at[u, pl.ds(start + off, n)])
                off += n
            plsc.subcore_barrier()

    @pl.when(c == 0)
    def _():
        do_relation((tb0, tb1, tb2, tb3), su_uc, sd_uc, NCP, sum_ce, 0,
                    cnt_ce)
        do_relation((ce0, ce1, ce2, ce3), su_cu, sd_cu, NTP, sum_tr, 0,
                    cnt_tr)

    @pl.when(c == 1)
    def _():
        do_relation((tb0, tb1, tb2, tb3), su_he, sd_he, NCP, sum_ce, 1,
                    cnt_ce)
        do_relation((ce0, ce1, ce2, ce3), su_eb, sd_eb, NTP, sum_tr, 1,
                    cnt_tr)


def _make_sc_agg(with_counts):
    out_type = [
        jax.ShapeDtypeStruct((2, 4, NCP, 32), jnp.float32),
        jax.ShapeDtypeStruct((2, 4, NTP, 32), jnp.float32),
    ]
    if with_counts:
        out_type += [
            jax.ShapeDtypeStruct((2, NCP, 32), jnp.float32),
            jax.ShapeDtypeStruct((2, NTP, 32), jnp.float32),
        ]
    return pl.kernel(
        functools.partial(_sc_body, with_counts),
        out_type=out_type,
        mesh=plsc.VectorSubcoreMesh(core_axis_name="c", subcore_axis_name="s"),
        compiler_params=pltpu.CompilerParams(use_tc_tiling_on_sc=False),
        scratch_types=[
            pltpu.VMEM_SHARED((NCP, 32), jnp.float32),
            pltpu.VMEM((CH, CHW), jnp.int32),
            pltpu.VMEM((CH, CHW), jnp.int32),
            pltpu.VMEM((CHW, 32), jnp.float32),
            pltpu.VMEM((CHW, 32), jnp.float32),
            pltpu.VMEM((256, 32), jnp.float32),
            pltpu.SemaphoreType.DMA,
            pltpu.SemaphoreType.DMA,
        ],
    )


def _mm(a, b):
    return jnp.dot(a, b, preferred_element_type=jnp.float32)


def _gru0(x, wih_t, bih, bhh):
    gi = _mm(x, wih_t) + bih
    r = jax.nn.sigmoid(gi[:, :HD] + bhh[:, :HD])
    z = jax.nn.sigmoid(gi[:, HD:2 * HD] + bhh[:, HD:2 * HD])
    n = jnp.tanh(gi[:, 2 * HD:] + r * bhh[:, 2 * HD:])
    return (1.0 - z) * n


def _norm(sum_blk, cnt_blk):
    y = jnp.concatenate([sum_blk[0], sum_blk[1], sum_blk[2], sum_blk[3]],
                        axis=1)
    return y * (1.0 / jnp.maximum(cnt_blk[:, :1], 1.0))


def _pre_body(ce_ref, r1_ref, b1_ref, out_ref):
    out_ref[...] = (_mm(ce_ref[0], r1_ref[...]) + b1_ref[...])[None]


def _t1_body(x_ref, scu_ref, ccu_ref, seb_ref, ceb_ref,
             gr1_ref, gb1_ref, gw12_ref, gw13_ref,
             lr_ref, lb_ref, lw2_ref, lw3_ref,
             lwih_ref, lbih_ref, lbhh_ref, lfcw_ref, lfcb_ref, fwl_ref,
             h0_ref, h1_ref, h2_ref, h3_ref, hld_ref):
    x = x_ref[...]
    y2 = _norm(scu_ref[0], ccu_ref[0])
    y3 = _norm(seb_ref[0], ceb_ref[0])
    h1 = jax.nn.relu(_mm(x, gr1_ref[...]) + gb1_ref[...]
                     + _mm(y2, gw12_ref[...]) + _mm(y3, gw13_ref[...]))
    h0_ref[...] = h1[:, 0:32]
    h1_ref[...] = h1[:, 32:64]
    h2_ref[...] = h1[:, 64:96]
    h3_ref[...] = h1[:, 96:128]
    nf = jax.nn.relu(_mm(x, lr_ref[...]) + lb_ref[...]
                     + _mm(y2, lw2_ref[...]) + _mm(y3, lw3_ref[...]))
    hl = _gru0(nf, lwih_ref[...], lbih_ref[...], lbhh_ref[...])
    lfeat = _mm(hl, lfcw_ref[...]) + lfcb_ref[...]
    hld_ref[...] = _mm(lfeat, fwl_ref[...])


def _ce1_body(pre_ref, s_ref, c_ref, w1_ref, h0_ref, h1_ref, h2_ref, h3_ref):
    y = _norm(s_ref[0], c_ref[0])
    h1 = jax.nn.relu(pre_ref[0] + _mm(y, w1_ref[0]))
    h0_ref[...] = h1[None, :, 0:32]
    h1_ref[...] = h1[None, :, 32:64]
    h2_ref[...] = h1[None, :, 64:96]
    h3_ref[...] = h1[None, :, 96:128]


def _t2_body(h0_ref, h1_ref, h2_ref, h3_ref, scu_ref, ccu_ref, seb_ref,
             ceb_ref, gr2_ref, gb2_ref, gw22_ref, gw23_ref, mx_ref):
    h = jnp.concatenate([h0_ref[...], h1_ref[...], h2_ref[...], h3_ref[...]],
                        axis=1)
    z2 = _norm(scu_ref[0], ccu_ref[0])
    z3 = _norm(seb_ref[0], ceb_ref[0])
    h2 = jax.nn.relu(_mm(h, gr2_ref[...]) + gb2_ref[...]
                     + _mm(z2, gw22_ref[...]) + _mm(z3, gw23_ref[...]))
    tile_max = jnp.broadcast_to(jnp.max(h2, axis=0, keepdims=True), (8, HD))

    @pl.when(pl.program_id(0) == 0)
    def _():
        mx_ref[...] = jnp.zeros_like(mx_ref)

    mx_ref[...] = jnp.maximum(mx_ref[...], tile_max)


def _ce2_body(h0_ref, h1_ref, h2_ref, h3_ref, s_ref, c_ref, gr2_ref, gb2_ref,
              w2_ref, mx_ref):
    h = jnp.concatenate([h0_ref[0], h1_ref[0], h2_ref[0], h3_ref[0]], axis=1)
    z = _norm(s_ref[0], c_ref[0])
    h2 = jax.nn.relu(_mm(h, gr2_ref[...]) + gb2_ref[...] + _mm(z, w2_ref[0]))
    tile_max = jnp.broadcast_to(jnp.max(h2, axis=0, keepdims=True), (8, HD))

    @pl.when((pl.program_id(0) == 0) & (pl.program_id(1) == 0))
    def _():
        mx_ref[...] = jnp.zeros_like(mx_ref)

    mx_ref[...] = jnp.maximum(mx_ref[...], tile_max)


def _fin_body(mt_ref, mce_ref, hld_ref,
              gwih_ref, gbih_ref, gbhh_ref, gfcw_ref, gfcb_ref, fwg_ref,
              fb_ref, out_ref):
    m = jnp.maximum(jnp.max(mt_ref[0], axis=0, keepdims=True),
                    jnp.max(mce_ref[0], axis=0, keepdims=True))
    hg = _gru0(m, gwih_ref[...], gbih_ref[...], gbhh_ref[...])
    gfeat = _mm(hg, gfcw_ref[...]) + gfcb_ref[...]
    sg = _mm(gfeat, fwg_ref[...]) + fb_ref[...]
    out_ref[...] = jax.nn.sigmoid(hld_ref[...] + sg[0, 0])


def kernel(trans_x, uc_src, uc_dst, he_src, he_dst, cu_src, cu_dst,
           eb_src, eb_dst, card_emb, email_emb,
           g_W1, g_R1, g_b1, g_W2, g_R2, g_b2, g_Wih, g_Whh, g_bih, g_bhh,
           g_fcW, g_fcb, l_W, l_R, l_b, l_Wih, l_Whh, l_bih, l_bhh,
           l_fcW, l_fcb, f_W, f_b):
    f32 = jnp.float32
    i32 = jnp.int32

    xt32 = trans_x.reshape(T, NT, 4, 32).transpose(0, 2, 1, 3)
    ce = jnp.stack([card_emb, email_emb])
    ce32 = ce.reshape(2, NC, 4, 32).transpose(2, 0, 1, 3).reshape(4, 2 * NC, 32)

    def prep_idx(src, dst, dump, src_off=0):
        pad_src = src_off + (jnp.arange(EPAD - E, dtype=i32) % 32)
        pad_dst = dump + (jnp.arange(EPAD - E, dtype=i32) % 32)
        sp = jnp.concatenate(
            [src.astype(i32) + src_off,
             jnp.broadcast_to(pad_src, (T, EPAD - E))], axis=1)
        dp = jnp.concatenate(
            [dst.astype(i32), jnp.broadcast_to(pad_dst, (T, EPAD - E))],
            axis=1)
        return (sp.reshape(T, NTILES, CH, CHW),
                dp.reshape(T, NTILES, CH, CHW))

    su_uc, sd_uc = prep_idx(uc_src, uc_dst, NC)
    su_he, sd_he = prep_idx(he_src, he_dst, NE)
    su_cu, sd_cu = prep_idx(cu_src, cu_dst, NT)
    su_eb, sd_eb = prep_idx(eb_src, eb_dst, NT, src_off=NC)

    zrows = jnp.zeros((256, 32), f32)
    orows = jnp.ones((CHW, 32), f32)

    gb1 = g_b1[None]; gb2 = g_b2[None]; lb = l_b[None]
    gw12, gw13 = g_W1[2], g_W1[3]
    gw22, gw23 = g_W2[2], g_W2[3]
    lw2, lw3 = l_W[2], l_W[3]
    lwih_t = l_Wih.T; gwih_t = g_Wih.T
    lbih = l_bih[None]; lbhh = l_bhh[None]
    gbih = g_bih[None]; gbhh = g_bhh[None]
    lfcw_t = l_fcW.T; gfcw_t = g_fcW.T
    lfcb = l_fcb[None]; gfcb = g_fcb[None]
    fwl = f_W[0, HD:][:, None]; fwg = f_W[0, :HD][:, None]
    fb = f_b[None]

    sc1 = _make_sc_agg(True)
    sc2 = _make_sc_agg(False)

    wspec = pl.BlockSpec((HD, HD), lambda i: (0, 0))
    bspec = pl.BlockSpec((1, HD), lambda i: (0, 0))
    b3spec = pl.BlockSpec((1, 3 * HD), lambda i: (0, 0))
    w3spec = pl.BlockSpec((HD, 3 * HD), lambda i: (0, 0))
    vspec = pl.BlockSpec((HD, 1), lambda i: (0, 0))
    xspec = pl.BlockSpec((RT, HD), lambda i: (i, 0))
    sum_ce_spec = pl.BlockSpec((1, 4, RT, 32), lambda u, i: (u, 0, i, 0))
    cnt_ce_spec = pl.BlockSpec((1, RT, 32), lambda u, i: (u, i, 0))
    sum_tr_spec = lambda u: pl.BlockSpec((1, 4, RT, 32),
                                         lambda i: (u, 0, i, 0))
    cnt_tr_spec = lambda u: pl.BlockSpec((1, RT, 32), lambda i: (u, i, 0))
    h32_spec = pl.BlockSpec((RT, 32), lambda i: (i, 0))
    hce_spec = pl.BlockSpec((1, RT, 32), lambda u, i: (u, i, 0))
    pre_spec = pl.BlockSpec((1, RT, HD), lambda u, i: (u, i, 0))
    wsel_spec = pl.BlockSpec((1, HD, HD), lambda u, i: (u, 0, 0))
    wfull2 = pl.BlockSpec((HD, HD), lambda u, i: (0, 0))
    bfull2 = pl.BlockSpec((1, HD), lambda u, i: (0, 0))
    mx_spec = pl.BlockSpec((8, HD), lambda i: (0, 0))
    mx2_spec = pl.BlockSpec((8, HD), lambda u, i: (0, 0))

    pre_ce = pl.pallas_call(
        _pre_body,
        grid=(2, NC // RT),
        in_specs=[pre_spec, wfull2, bfull2],
        out_specs=pre_spec,
        out_shape=jax.ShapeDtypeStruct((2, NC, HD), f32),
    )(ce, g_R1, gb1)

    t1_call = pl.pallas_call(
        _t1_body,
        grid=(NT // RT,),
        in_specs=[xspec, sum_tr_spec(0), cnt_tr_spec(0), sum_tr_spec(1),
                  cnt_tr_spec(1), wspec, bspec, wspec, wspec,
                  wspec, bspec, wspec, wspec,
                  w3spec, b3spec, b3spec, wspec, bspec, vspec],
        out_specs=[h32_spec, h32_spec, h32_spec, h32_spec,
                   pl.BlockSpec((RT, 1), lambda i: (i, 0))],
        out_shape=[jax.ShapeDtypeStruct((NT, 32), f32)] * 4
        + [jax.ShapeDtypeStruct((NT, 1), f32)],
    )

    ce1_call = pl.pallas_call(
        _ce1_body,
        grid=(2, NC // RT),
        in_specs=[pre_spec, sum_ce_spec, cnt_ce_spec, wsel_spec],
        out_specs=[hce_spec] * 4,
        out_shape=[jax.ShapeDtypeStruct((2, NC, 32), f32)] * 4,
    )

    t2_call = pl.pallas_call(
        _t2_body,
        grid=(NT // RT,),
        in_specs=[h32_spec, h32_spec, h32_spec, h32_spec,
                  sum_tr_spec(0), cnt_tr_spec(0), sum_tr_spec(1),
                  cnt_tr_spec(1), wspec, bspec, wspec, wspec],
        out_specs=mx_spec,
        out_shape=jax.ShapeDtypeStruct((8, HD), f32),
        compiler_params=pltpu.CompilerParams(
            dimension_semantics=("arbitrary",)),
    )

    ce2_call = pl.pallas_call(
        _ce2_body,
        grid=(2, NC // RT),
        in_specs=[hce_spec, hce_spec, hce_spec, hce_spec,
                  sum_ce_spec, cnt_ce_spec, wfull2, bfull2, wsel_spec],
        out_specs=mx2_spec,
        out_shape=jax.ShapeDtypeStruct((8, HD), f32),
        compiler_params=pltpu.CompilerParams(
            dimension_semantics=("arbitrary", "arbitrary")),
    )

    mts, mces, hlds = [], [], []
    for t in range(T):
        tb = [xt32[t, b] for b in range(4)]
        cearg = [ce32[b] for b in range(4)]
        idx = (su_uc[t], sd_uc[t], su_he[t], sd_he[t],
               su_cu[t], sd_cu[t], su_eb[t], sd_eb[t])
        sum_ce1, sum_tr1, cnt_ce, cnt_tr = sc1(
            *tb, *cearg, *idx, zrows, orows)
        h1t = t1_call(trans_x[t], sum_tr1, cnt_tr, sum_tr1, cnt_tr,
                      g_R1, gb1, gw12, gw13, l_R, lb, lw2, lw3,
                      lwih_t, lbih, lbhh, lfcw_t, lfcb, fwl)
        h1t_b, hld = h1t[:4], h1t[4]
        h1ce_b = ce1_call(pre_ce, sum_ce1, cnt_ce, g_W1)
        h1ce_flat = [a.reshape(2 * NC, 32) for a in h1ce_b]
        sum_ce2, sum_tr2 = sc2(*h1t_b, *h1ce_flat, *idx, zrows, orows)
        mt = t2_call(*h1t_b, sum_tr2, cnt_tr, sum_tr2, cnt_tr,
                     g_R2, gb2, gw22, gw23)
        mce = ce2_call(*h1ce_b, sum_ce2, cnt_ce, g_R2, gb2, g_W2)
        mts.append(mt); mces.append(mce); hlds.append(hld)

    mt_all = jnp.stack(mts)
    mce_all = jnp.stack(mces)
    hld_all = jnp.stack(hlds)

    preds = pl.pallas_call(
        _fin_body,
        grid=(T, NT // RT),
        in_specs=[pl.BlockSpec((1, 8, HD), lambda t, i: (t, 0, 0)),
                  pl.BlockSpec((1, 8, HD), lambda t, i: (t, 0, 0)),
                  pl.BlockSpec((1, RT, 1), lambda t, i: (t, i, 0)),
                  pl.BlockSpec((HD, 3 * HD), lambda t, i: (0, 0)),
                  pl.BlockSpec((1, 3 * HD), lambda t, i: (0, 0)),
                  pl.BlockSpec((1, 3 * HD), lambda t, i: (0, 0)),
                  pl.BlockSpec((HD, HD), lambda t, i: (0, 0)),
                  pl.BlockSpec((1, HD), lambda t, i: (0, 0)),
                  pl.BlockSpec((HD, 1), lambda t, i: (0, 0)),
                  pl.BlockSpec((1, 1), lambda t, i: (0, 0))],
        out_specs=pl.BlockSpec((1, RT, 1), lambda t, i: (t, i, 0)),
        out_shape=jax.ShapeDtypeStruct((T, NT, 1), f32),
    )(mt_all, mce_all, hld_all, gwih_t, gbih, gbhh, gfcw_t, gfcb, fwg, fb)

    return preds.reshape(-1)
